# Initial kernel scaffold; baseline (speedup 1.0000x reference)
import jax
import jax.numpy as jnp
from jax.experimental import pallas as pl


def kernel(data, indices, updates):
    raise NotImplementedError("write your pallas kernel here")


# speedup vs baseline: 27.4296x; 27.4296x over previous
import functools

import jax
import jax.numpy as jnp
from jax import lax
from jax.experimental import pallas as pl
from jax.experimental.pallas import tpu as pltpu
from jax.experimental.pallas import tpu_sc as plsc

M_TOTAL = 1000000
N_TOTAL = 4194304

NC = 2
NS = 16
NW = NC * NS

E_PER_TILE = N_TOTAL // NW
ROW = 128
K_ROWS = E_PER_TILE // ROW
RC = 16
NCHUNK = K_ROWS // RC

M_PAD = 1000448
TCHUNK = M_PAD // NS
PAD2D = M_PAD // 128


def _sc_body(data_hbm, zeros_hbm, idx_hbm, upd_hbm, p0_hbm, p1_hbm,
             idx_v, upd_v, stage_v, acc_sh):
    c = lax.axis_index("c")
    s = lax.axis_index("s")
    w = c * NS + s
    tb = s * TCHUNK

    @pl.when(c == 0)
    def _():
        pltpu.sync_copy(data_hbm.at[pl.ds(tb, TCHUNK)], stage_v)

    @pl.when(c != 0)
    def _():
        pltpu.sync_copy(zeros_hbm.at[pl.ds(tb, TCHUNK)], stage_v)

    pltpu.sync_copy(stage_v, acc_sh.at[pl.ds(tb, TCHUNK)])

    plsc.subcore_barrier()

    def chunk(ch, carry):
        r0 = ch * RC
        pltpu.sync_copy(idx_hbm.at[w, pl.ds(r0, RC)], idx_v)
        pltpu.sync_copy(upd_hbm.at[w, pl.ds(r0, RC)], upd_v)
        for r in range(RC):
            pltpu.sync_copy(upd_v.at[r], acc_sh.at[idx_v.at[r]], add=True)
        return carry

    lax.fori_loop(0, NCHUNK, chunk, 0)

    plsc.subcore_barrier()

    pltpu.sync_copy(acc_sh.at[pl.ds(tb, TCHUNK)], stage_v)

    @pl.when(c == 0)
    def _():
        pltpu.sync_copy(stage_v, p0_hbm.at[pl.ds(tb, TCHUNK)])

    @pl.when(c != 0)
    def _():
        pltpu.sync_copy(stage_v, p1_hbm.at[pl.ds(tb, TCHUNK)])


_sc_scatter = functools.partial(
    pl.kernel,
    out_type=(jax.ShapeDtypeStruct((M_PAD,), jnp.float32),
              jax.ShapeDtypeStruct((M_PAD,), jnp.float32)),
    mesh=plsc.VectorSubcoreMesh(core_axis_name="c", subcore_axis_name="s"),
    scratch_types=(
        pltpu.VMEM((RC, ROW), jnp.int32),
        pltpu.VMEM((RC, ROW), jnp.float32),
        pltpu.VMEM((TCHUNK,), jnp.float32),
        pltpu.VMEM_SHARED((M_PAD,), jnp.float32),
    ),
)(_sc_body)


def _combine_body(a_ref, b_ref, o_ref):
    o_ref[...] = a_ref[...] + b_ref[...]


def _combine(a, b):
    return pl.pallas_call(
        _combine_body,
        out_shape=jax.ShapeDtypeStruct((PAD2D, 128), jnp.float32),
    )(a, b)


@jax.jit
def kernel(data, indices, updates):
    data_pad = jnp.pad(data, (0, M_PAD - M_TOTAL))
    zeros = jnp.zeros((M_PAD,), jnp.float32)
    idx3 = indices.astype(jnp.int32).reshape(NW, K_ROWS, ROW)
    upd3 = updates.reshape(NW, K_ROWS, ROW)
    p0, p1 = _sc_scatter(data_pad, zeros, idx3, upd3)
    out2d = _combine(p0.reshape(PAD2D, 128), p1.reshape(PAD2D, 128))
    return out2d.reshape(-1)[:M_TOTAL]
